# baseline (speedup 1.0000x reference)
import functools

import jax
import jax.numpy as jnp
import numpy as np
from jax import lax
from jax.experimental import pallas as pl
from jax.experimental.pallas import tpu as pltpu
from jax.experimental.pallas import tpu_sc as plsc

NM = 64
B = 16384
H = 64
FI = 6
FO = 3
FP = 16
BLK = 256
CAP = B + NM * BLK
NBLK = CAP // BLK
RB = 512
NB = B // RB
NW = 32
CHUNK = B // NW


def _onehot(sel):
    selc = jnp.minimum(jnp.maximum(sel, 0), NM - 1)
    m_iota = lax.broadcasted_iota(jnp.int32, (NM, RB), 0)
    return (m_iota == selc).astype(jnp.float32)


def _hist_body(x0_ref, x2_ref, sel_ref, po_ref, be_ref, cnt0):
    j = pl.program_id(0)
    f32 = jnp.float32

    ang = jnp.arctan2(x2_ref[0], x0_ref[0])
    ang = jnp.fmod(ang + 2 * np.pi, 2 * np.pi) / (2 * np.pi) * NM
    sel = jnp.floor(ang).astype(jnp.int32)
    sel_ref[0] = sel

    onehot = _onehot(sel)
    rs = jnp.sum(onehot, axis=1, keepdims=True)

    @pl.when(j == 0)
    def _init():
        cnt0[...] = jnp.zeros((NM, 128), f32)

    cnt0[...] += jnp.broadcast_to(rs, (NM, 128))

    @pl.when(j == NB - 1)
    def _finish():
        c = cnt0[...]
        pc = jnp.ceil(c / BLK) * BLK
        ii = lax.broadcasted_iota(jnp.int32, (NM, NM), 0)
        jj = lax.broadcasted_iota(jnp.int32, (NM, NM), 1)
        tri = (jj < ii).astype(f32)
        po = jnp.dot(tri, pc, preferred_element_type=f32)
        po_ref[...] = po
        pe = po + pc
        jl = lax.broadcasted_iota(jnp.int32, (NM, 128), 1).astype(f32) * float(BLK)
        mask = (po <= jl) & (jl < pe)
        mvals = lax.broadcasted_iota(jnp.int32, (NM, 128), 0).astype(f32)
        be = jnp.sum(jnp.where(mask, mvals, 0.0), axis=0, keepdims=True)
        be_ref[...] = be.astype(jnp.int32)


def _dest_body(sel_ref, po_ref, dest_ref, cnt1):
    j = pl.program_id(0)
    f32 = jnp.float32

    @pl.when(j == 0)
    def _init():
        cnt1[...] = jnp.zeros((NM, 128), f32)

    sel = sel_ref[0]
    onehot = _onehot(sel)
    rs = jnp.sum(onehot, axis=1, keepdims=True)
    ii = lax.broadcasted_iota(jnp.int32, (RB, RB), 0)
    jj = lax.broadcasted_iota(jnp.int32, (RB, RB), 1)
    tri = (ii < jj).astype(f32)
    csum = jnp.dot(onehot, tri, preferred_element_type=f32)
    add = po_ref[:, 0:1] + cnt1[...][:, 0:1]
    destf = jnp.sum(onehot * (csum + add), axis=0, keepdims=True)
    dest_ref[0] = destf.astype(jnp.int32)
    cnt1[...] += jnp.broadcast_to(rs, (NM, 128))


def _route(inputs):
    f32 = jnp.float32
    x0r = inputs[:, 0].reshape(NB, 1, RB)
    x2r = inputs[:, 2].reshape(NB, 1, RB)
    spec = pl.BlockSpec((1, 1, RB), lambda j: (j, 0, 0))
    cspec = lambda r: pl.BlockSpec((r, 128), lambda j: (0, 0))
    sel3, po, be2 = pl.pallas_call(
        _hist_body,
        grid=(NB,),
        in_specs=[spec, spec],
        out_specs=[spec, cspec(NM), cspec(1)],
        out_shape=[
            jax.ShapeDtypeStruct((NB, 1, RB), jnp.int32),
            jax.ShapeDtypeStruct((NM, 128), f32),
            jax.ShapeDtypeStruct((1, 128), jnp.int32),
        ],
        scratch_shapes=[pltpu.VMEM((NM, 128), f32)],
    )(x0r, x2r)
    dest3 = pl.pallas_call(
        _dest_body,
        grid=(NB,),
        in_specs=[spec, cspec(NM)],
        out_specs=spec,
        out_shape=jax.ShapeDtypeStruct((NB, 1, RB), jnp.int32),
        scratch_shapes=[pltpu.VMEM((NM, 128), f32)],
    )(sel3, po)
    return sel3.reshape(B), dest3.reshape(B), be2.reshape(NBLK)


@functools.cache
def _sc_kernels():
    mesh = plsc.VectorSubcoreMesh(core_axis_name="c", subcore_axis_name="s")
    scratch = [
        pltpu.VMEM((4, 128), jnp.int32),
        pltpu.VMEM((CHUNK, FP), jnp.float32),
        pltpu.SemaphoreType.DMA,
    ]

    cparams = pltpu.CompilerParams(use_tc_tiling_on_sc=False)

    @functools.partial(
        pl.kernel, mesh=mesh,
        out_type=jax.ShapeDtypeStruct((CAP, FP), jnp.float32),
        scratch_types=scratch,
        compiler_params=cparams,
    )
    def scatter_k(x_hbm, idx_hbm, out_hbm, idx_v, rows_v, sem):
        wid = lax.axis_index("s") * 2 + lax.axis_index("c")
        base = wid * CHUNK
        pltpu.sync_copy(idx_hbm.at[wid], idx_v)
        pltpu.sync_copy(x_hbm.at[pl.ds(base, CHUNK)], rows_v)
        for j in range(4):
            pltpu.async_copy(rows_v.at[pl.ds(j * 128, 128)],
                             out_hbm.at[idx_v.at[j]], sem).wait()

    @functools.partial(
        pl.kernel, mesh=mesh,
        out_type=jax.ShapeDtypeStruct((B, FP), jnp.float32),
        scratch_types=scratch,
        compiler_params=cparams,
    )
    def gather_k(ys_hbm, idx_hbm, out_hbm, idx_v, rows_v, sem):
        wid = lax.axis_index("s") * 2 + lax.axis_index("c")
        base = wid * CHUNK
        pltpu.sync_copy(idx_hbm.at[wid], idx_v)
        for j in range(4):
            pltpu.async_copy(ys_hbm.at[idx_v.at[j]],
                             rows_v.at[pl.ds(j * 128, 128)], sem).wait()
        pltpu.sync_copy(rows_v, out_hbm.at[pl.ds(base, CHUNK)])

    return scatter_k, gather_k


def _scatter_rows(xp, dest3):
    return _sc_kernels()[0](xp, dest3)


def _gather_rows(ys, dest3):
    return _sc_kernels()[1](ys, dest3)


CH = 4


def _mlp_body(be_s, *refs):
    f32 = jnp.float32
    i = pl.program_id(0)
    xs_ref = refs[0]
    out_ref = refs[1 + 10 * CH]
    w0bd, w1bd, w2bd, w3bd, w4bd = refs[2 + 10 * CH:]
    chains = [refs[1 + 10 * k:11 + 10 * k] for k in range(CH)]

    @pl.when(i == 0)
    def _zero():
        w0bd[...] = jnp.zeros((CH * FP, CH * H), f32)
        w1bd[...] = jnp.zeros((CH * H, CH * H), f32)
        w2bd[...] = jnp.zeros((CH * H, CH * H), f32)
        w3bd[...] = jnp.zeros((CH * H, CH * H), f32)
        w4bd[...] = jnp.zeros((CH * H, CH * FP), f32)

    for k, (w0, b0, w1, b1, w2, b2, w3, b3, w4, b4) in enumerate(chains):
        w0bd[pl.ds(k * FP, FP), pl.ds(k * H, H)] = w0[0]
        w1bd[pl.ds(k * H, H), pl.ds(k * H, H)] = w1[0]
        w2bd[pl.ds(k * H, H), pl.ds(k * H, H)] = w2[0]
        w3bd[pl.ds(k * H, H), pl.ds(k * H, H)] = w3[0]
        w4bd[pl.ds(k * H, H), pl.ds(k * FP, FP)] = w4[0]

    x0 = jnp.concatenate(
        [xs_ref[pl.ds(k * BLK, BLK), :] for k in range(CH)], axis=1)
    bcat = lambda idx: jnp.concatenate([c[idx][0] for c in chains], axis=1)
    y = jnp.maximum(jnp.dot(x0, w0bd[...], preferred_element_type=f32)
                    + bcat(1), 0.0)
    y = jnp.maximum(jnp.dot(y, w1bd[...], preferred_element_type=f32)
                    + bcat(3), 0.0)
    y = jnp.maximum(jnp.dot(y, w2bd[...], preferred_element_type=f32)
                    + bcat(5), 0.0)
    y = jnp.maximum(jnp.dot(y, w3bd[...], preferred_element_type=f32)
                    + bcat(7), 0.0)
    y4 = jnp.dot(y, w4bd[...], preferred_element_type=f32) + bcat(9)
    for k in range(CH):
        out_ref[pl.ds(k * BLK, BLK), :] = y4[:, k * FP:(k + 1) * FP]


def _expert_mlp(xs, be, w0t, b0r, w1t, b1r, w2t, b2r, w3t, b3r, w4t, b4r):
    f32 = jnp.float32

    def wspec(r, c, k):
        return pl.BlockSpec((1, r, c),
                            lambda i, be_s, k=k: (be_s[CH * i + k], 0, 0))

    def bspec(c, k):
        return pl.BlockSpec((1, 1, c),
                            lambda i, be_s, k=k: (be_s[CH * i + k], 0, 0))

    in_specs = [pl.BlockSpec((CH * BLK, FP), lambda i, be_s: (i, 0))]
    for k in range(CH):
        in_specs += [
            wspec(FP, H, k), bspec(H, k),
            wspec(H, H, k), bspec(H, k),
            wspec(H, H, k), bspec(H, k),
            wspec(H, H, k), bspec(H, k),
            wspec(H, FP, k), bspec(FP, k),
        ]
    grid_spec = pltpu.PrefetchScalarGridSpec(
        num_scalar_prefetch=1,
        grid=(NBLK // CH,),
        in_specs=in_specs,
        out_specs=pl.BlockSpec((CH * BLK, FP), lambda i, be_s: (i, 0)),
        scratch_shapes=[
            pltpu.VMEM((CH * FP, CH * H), f32),
            pltpu.VMEM((CH * H, CH * H), f32),
            pltpu.VMEM((CH * H, CH * H), f32),
            pltpu.VMEM((CH * H, CH * H), f32),
            pltpu.VMEM((CH * H, CH * FP), f32),
        ],
    )
    ws = (w0t, b0r, w1t, b1r, w2t, b2r, w3t, b3r, w4t, b4r)
    return pl.pallas_call(
        _mlp_body,
        grid_spec=grid_spec,
        out_shape=jax.ShapeDtypeStruct((CAP, FP), f32),
    )(be, xs, *(ws * CH))


def kernel(inputs, W0, b0, W1, b1, W2, b2, W3, b3, W4, b4):
    f32 = jnp.float32
    xp = jnp.zeros((B, FP), f32).at[:, :FI].set(inputs)
    w0t = jnp.zeros((NM, FP, H), f32).at[:, :FI, :].set(
        jnp.transpose(W0, (0, 2, 1)))
    w1t = jnp.transpose(W1, (0, 2, 1))
    w2t = jnp.transpose(W2, (0, 2, 1))
    w3t = jnp.transpose(W3, (0, 2, 1))
    w4t = jnp.zeros((NM, H, FP), f32).at[:, :, :FO].set(
        jnp.transpose(W4, (0, 2, 1)))
    b4p = jnp.zeros((NM, FP), f32).at[:, :FO].set(b4)
    b0r, b1r, b2r, b3r = (b[:, None, :] for b in (b0, b1, b2, b3))
    b4r = b4p[:, None, :]

    sel, dest, be = _route(inputs)
    dest3 = dest.reshape(NW, 4, 128)
    xs = _scatter_rows(xp, dest3)
    ys = _expert_mlp(xs, be, w0t, b0r, w1t, b1r, w2t, b2r, w3t, b3r,
                     w4t, b4r)
    out = _gather_rows(ys, dest3)

    model_output = out[:, :FO]
    top_outputs = model_output[:, None, :]
    selection_logits = jnp.ones((B, NM), f32)
    selection_probabilities = jnp.full((B, NM), 1.0 / NM, f32)
    return (model_output, top_outputs, sel,
            selection_logits, selection_probabilities)


# speedup vs baseline: 1.0031x; 1.0031x over previous
import functools

import jax
import jax.numpy as jnp
import numpy as np
from jax import lax
from jax.experimental import pallas as pl
from jax.experimental.pallas import tpu as pltpu
from jax.experimental.pallas import tpu_sc as plsc

NM = 64
B = 16384
H = 64
FI = 6
FO = 3
FP = 16
BLK = 256
CAP = B + NM * BLK
NBLK = CAP // BLK
RB = 512
NB = B // RB
NW = 32
CHUNK = B // NW


def _onehot(sel):
    selc = jnp.minimum(jnp.maximum(sel, 0), NM - 1)
    m_iota = lax.broadcasted_iota(jnp.int32, (NM, RB), 0)
    return (m_iota == selc).astype(jnp.float32)


def _route_body(x0_ref, x2_ref, sd_ref, be_ref, cnt0, po_s, sel_s):
    p = pl.program_id(0)
    j = pl.program_id(1)
    f32 = jnp.float32

    @pl.when(p == 0)
    def _phase0():
        ang = jnp.arctan2(x2_ref[0], x0_ref[0])
        ang = jnp.fmod(ang + 2 * np.pi, 2 * np.pi) / (2 * np.pi) * NM
        sel = jnp.floor(ang).astype(jnp.int32)
        sd_ref[0] = sel
        sel_s[pl.ds(j, 1), :] = sel
        onehot = _onehot(sel)
        rs = jnp.sum(onehot, axis=1, keepdims=True)

        @pl.when(j == 0)
        def _init():
            cnt0[...] = jnp.zeros((NM, 128), f32)

        cnt0[...] += jnp.broadcast_to(rs, (NM, 128))

        @pl.when(j == NB - 1)
        def _finish():
            c = cnt0[...]
            pc = jnp.ceil(c / BLK) * BLK
            ii = lax.broadcasted_iota(jnp.int32, (NM, NM), 0)
            jj = lax.broadcasted_iota(jnp.int32, (NM, NM), 1)
            tri = (jj < ii).astype(f32)
            po = jnp.dot(tri, pc, preferred_element_type=f32)
            po_s[...] = po
            pe = po + pc
            jl = lax.broadcasted_iota(jnp.int32, (NM, 128), 1).astype(f32) * float(BLK)
            mask = (po <= jl) & (jl < pe)
            mvals = lax.broadcasted_iota(jnp.int32, (NM, 128), 0).astype(f32)
            be = jnp.sum(jnp.where(mask, mvals, 0.0), axis=0, keepdims=True)
            be_ref[...] = be.astype(jnp.int32)
            cnt0[...] = jnp.zeros((NM, 128), f32)

    @pl.when(p == 1)
    def _phase1():
        sel = sel_s[pl.ds(j, 1), :]
        onehot = _onehot(sel)
        rs = jnp.sum(onehot, axis=1, keepdims=True)
        ii = lax.broadcasted_iota(jnp.int32, (RB, RB), 0)
        jj = lax.broadcasted_iota(jnp.int32, (RB, RB), 1)
        tri = (ii < jj).astype(f32)
        csum = jnp.dot(onehot, tri, preferred_element_type=f32)
        add = po_s[...][:, 0:1] + cnt0[...][:, 0:1]
        destf = jnp.sum(onehot * (csum + add), axis=0, keepdims=True)
        sd_ref[0] = destf.astype(jnp.int32)
        cnt0[...] += jnp.broadcast_to(rs, (NM, 128))


def _route(inputs):
    f32 = jnp.float32
    x0r = inputs[:, 0].reshape(NB, 1, RB)
    x2r = inputs[:, 2].reshape(NB, 1, RB)
    inspec = pl.BlockSpec((1, 1, RB), lambda p, j: (j, 0, 0))
    sd3, be2 = pl.pallas_call(
        _route_body,
        grid=(2, NB),
        in_specs=[inspec, inspec],
        out_specs=[
            pl.BlockSpec((1, 1, RB), lambda p, j: (p * NB + j, 0, 0)),
            pl.BlockSpec((1, 128), lambda p, j: (0, 0)),
        ],
        out_shape=[
            jax.ShapeDtypeStruct((2 * NB, 1, RB), jnp.int32),
            jax.ShapeDtypeStruct((1, 128), jnp.int32),
        ],
        scratch_shapes=[
            pltpu.VMEM((NM, 128), f32),
            pltpu.VMEM((NM, 128), f32),
            pltpu.VMEM((NB, RB), jnp.int32),
        ],
    )(x0r, x2r)
    return (sd3[:NB].reshape(B), sd3[NB:].reshape(B), be2.reshape(NBLK))


@functools.cache
def _sc_kernels():
    mesh = plsc.VectorSubcoreMesh(core_axis_name="c", subcore_axis_name="s")
    scratch = [
        pltpu.VMEM((4, 128), jnp.int32),
        pltpu.VMEM((CHUNK, FP), jnp.float32),
        pltpu.SemaphoreType.DMA,
        pltpu.SemaphoreType.DMA,
        pltpu.SemaphoreType.DMA,
    ]

    cparams = pltpu.CompilerParams(use_tc_tiling_on_sc=False)

    @functools.partial(
        pl.kernel, mesh=mesh,
        out_type=jax.ShapeDtypeStruct((CAP, FP), jnp.float32),
        scratch_types=scratch,
        compiler_params=cparams,
    )
    def scatter_k(x_hbm, idx_hbm, out_hbm, idx_v, rows_v, sem_a, sem_b,
                  sem_c):
        wid = lax.axis_index("s") * 2 + lax.axis_index("c")
        base = wid * CHUNK
        cp_i = pltpu.async_copy(idx_hbm.at[wid], idx_v, sem_a)
        cp_x = pltpu.async_copy(x_hbm.at[pl.ds(base, CHUNK)], rows_v, sem_b)
        cp_i.wait()
        cp_x.wait()
        cps = [pltpu.async_copy(rows_v.at[pl.ds(j * 128, 128)],
                                out_hbm.at[idx_v.at[j]], sem_c)
               for j in range(4)]
        for cp in cps:
            cp.wait()

    @functools.partial(
        pl.kernel, mesh=mesh,
        out_type=jax.ShapeDtypeStruct((B, FP), jnp.float32),
        scratch_types=scratch,
        compiler_params=cparams,
    )
    def gather_k(ys_hbm, idx_hbm, out_hbm, idx_v, rows_v, sem_a, sem_b,
                 sem_c):
        wid = lax.axis_index("s") * 2 + lax.axis_index("c")
        base = wid * CHUNK
        pltpu.async_copy(idx_hbm.at[wid], idx_v, sem_a).wait()
        cps = [pltpu.async_copy(ys_hbm.at[idx_v.at[j]],
                                rows_v.at[pl.ds(j * 128, 128)], sem_b)
               for j in range(4)]
        for cp in cps:
            cp.wait()
        pltpu.sync_copy(rows_v, out_hbm.at[pl.ds(base, CHUNK)])

    return scatter_k, gather_k


def _scatter_rows(xp, dest3):
    return _sc_kernels()[0](xp, dest3)


def _gather_rows(ys, dest3):
    return _sc_kernels()[1](ys, dest3)


CH = 4


def _mlp_body(be_s, *refs):
    f32 = jnp.float32
    i = pl.program_id(0)
    xs_ref = refs[0]
    out_ref = refs[1 + 10 * CH]
    w0bd, w1bd, w2bd, w3bd, w4bd = refs[2 + 10 * CH:]
    chains = [refs[1 + 10 * k:11 + 10 * k] for k in range(CH)]

    @pl.when(i == 0)
    def _zero():
        w0bd[...] = jnp.zeros((CH * FP, CH * H), f32)
        w1bd[...] = jnp.zeros((CH * H, CH * H), f32)
        w2bd[...] = jnp.zeros((CH * H, CH * H), f32)
        w3bd[...] = jnp.zeros((CH * H, CH * H), f32)
        w4bd[...] = jnp.zeros((CH * H, CH * FP), f32)

    for k, (w0, b0, w1, b1, w2, b2, w3, b3, w4, b4) in enumerate(chains):
        w0bd[pl.ds(k * FP, FP), pl.ds(k * H, H)] = w0[0]
        w1bd[pl.ds(k * H, H), pl.ds(k * H, H)] = w1[0]
        w2bd[pl.ds(k * H, H), pl.ds(k * H, H)] = w2[0]
        w3bd[pl.ds(k * H, H), pl.ds(k * H, H)] = w3[0]
        w4bd[pl.ds(k * H, H), pl.ds(k * FP, FP)] = w4[0]

    x0 = jnp.concatenate(
        [xs_ref[pl.ds(k * BLK, BLK), :] for k in range(CH)], axis=1)
    bcat = lambda idx: jnp.concatenate([c[idx][0] for c in chains], axis=1)
    y = jnp.maximum(jnp.dot(x0, w0bd[...], preferred_element_type=f32)
                    + bcat(1), 0.0)
    y = jnp.maximum(jnp.dot(y, w1bd[...], preferred_element_type=f32)
                    + bcat(3), 0.0)
    y = jnp.maximum(jnp.dot(y, w2bd[...], preferred_element_type=f32)
                    + bcat(5), 0.0)
    y = jnp.maximum(jnp.dot(y, w3bd[...], preferred_element_type=f32)
                    + bcat(7), 0.0)
    y4 = jnp.dot(y, w4bd[...], preferred_element_type=f32) + bcat(9)
    for k in range(CH):
        out_ref[pl.ds(k * BLK, BLK), :] = y4[:, k * FP:(k + 1) * FP]


def _expert_mlp(xs, be, w0t, b0r, w1t, b1r, w2t, b2r, w3t, b3r, w4t, b4r):
    f32 = jnp.float32

    def wspec(r, c, k):
        return pl.BlockSpec((1, r, c),
                            lambda i, be_s, k=k: (be_s[CH * i + k], 0, 0))

    def bspec(c, k):
        return pl.BlockSpec((1, 1, c),
                            lambda i, be_s, k=k: (be_s[CH * i + k], 0, 0))

    in_specs = [pl.BlockSpec((CH * BLK, FP), lambda i, be_s: (i, 0))]
    for k in range(CH):
        in_specs += [
            wspec(FP, H, k), bspec(H, k),
            wspec(H, H, k), bspec(H, k),
            wspec(H, H, k), bspec(H, k),
            wspec(H, H, k), bspec(H, k),
            wspec(H, FP, k), bspec(FP, k),
        ]
    grid_spec = pltpu.PrefetchScalarGridSpec(
        num_scalar_prefetch=1,
        grid=(NBLK // CH,),
        in_specs=in_specs,
        out_specs=pl.BlockSpec((CH * BLK, FP), lambda i, be_s: (i, 0)),
        scratch_shapes=[
            pltpu.VMEM((CH * FP, CH * H), f32),
            pltpu.VMEM((CH * H, CH * H), f32),
            pltpu.VMEM((CH * H, CH * H), f32),
            pltpu.VMEM((CH * H, CH * H), f32),
            pltpu.VMEM((CH * H, CH * FP), f32),
        ],
    )
    ws = (w0t, b0r, w1t, b1r, w2t, b2r, w3t, b3r, w4t, b4r)
    return pl.pallas_call(
        _mlp_body,
        grid_spec=grid_spec,
        out_shape=jax.ShapeDtypeStruct((CAP, FP), f32),
    )(be, xs, *(ws * CH))


def kernel(inputs, W0, b0, W1, b1, W2, b2, W3, b3, W4, b4):
    f32 = jnp.float32
    xp = jnp.zeros((B, FP), f32).at[:, :FI].set(inputs)
    w0t = jnp.zeros((NM, FP, H), f32).at[:, :FI, :].set(
        jnp.transpose(W0, (0, 2, 1)))
    w1t = jnp.transpose(W1, (0, 2, 1))
    w2t = jnp.transpose(W2, (0, 2, 1))
    w3t = jnp.transpose(W3, (0, 2, 1))
    w4t = jnp.zeros((NM, H, FP), f32).at[:, :, :FO].set(
        jnp.transpose(W4, (0, 2, 1)))
    b4p = jnp.zeros((NM, FP), f32).at[:, :FO].set(b4)
    b0r, b1r, b2r, b3r = (b[:, None, :] for b in (b0, b1, b2, b3))
    b4r = b4p[:, None, :]

    sel, dest, be = _route(inputs)
    dest3 = dest.reshape(NW, 4, 128)
    xs = _scatter_rows(xp, dest3)
    ys = _expert_mlp(xs, be, w0t, b0r, w1t, b1r, w2t, b2r, w3t, b3r,
                     w4t, b4r)
    out = _gather_rows(ys, dest3)

    model_output = out[:, :FO]
    top_outputs = model_output[:, None, :]
    selection_logits = jnp.ones((B, NM), f32)
    selection_probabilities = jnp.full((B, NM), 1.0 / NM, f32)
    return (model_output, top_outputs, sel,
            selection_logits, selection_probabilities)


# speedup vs baseline: 1.0307x; 1.0275x over previous
import functools

import jax
import jax.numpy as jnp
import numpy as np
from jax import lax
from jax.experimental import pallas as pl
from jax.experimental.pallas import tpu as pltpu
from jax.experimental.pallas import tpu_sc as plsc

NM = 64
B = 16384
H = 64
FI = 6
FO = 3
FP = 16
BLK = 256
CAP = B + NM * BLK
NBLK = CAP // BLK
RB = 512
NB = B // RB
NW = 32
CHUNK = B // NW


def _onehot(sel):
    selc = jnp.minimum(jnp.maximum(sel, 0), NM - 1)
    m_iota = lax.broadcasted_iota(jnp.int32, (NM, RB), 0)
    return (m_iota == selc).astype(jnp.float32)


def _hist_body(x0_ref, x2_ref, sel_ref, po_ref, be_ref, cnt0):
    j = pl.program_id(0)
    f32 = jnp.float32

    ang = jnp.arctan2(x2_ref[0], x0_ref[0])
    ang = jnp.fmod(ang + 2 * np.pi, 2 * np.pi) / (2 * np.pi) * NM
    sel = jnp.floor(ang).astype(jnp.int32)
    sel_ref[0] = sel

    onehot = _onehot(sel)
    rs = jnp.sum(onehot, axis=1, keepdims=True)

    @pl.when(j == 0)
    def _init():
        cnt0[...] = jnp.zeros((NM, 128), f32)

    cnt0[...] += jnp.broadcast_to(rs, (NM, 128))

    @pl.when(j == NB - 1)
    def _finish():
        c = cnt0[...]
        pc = jnp.ceil(c / BLK) * BLK
        ii = lax.broadcasted_iota(jnp.int32, (NM, NM), 0)
        jj = lax.broadcasted_iota(jnp.int32, (NM, NM), 1)
        tri = (jj < ii).astype(f32)
        po = jnp.dot(tri, pc, preferred_element_type=f32)
        po_ref[...] = po
        pe = po + pc
        jl = lax.broadcasted_iota(jnp.int32, (NM, 128), 1).astype(f32) * float(BLK)
        mask = (po <= jl) & (jl < pe)
        mvals = lax.broadcasted_iota(jnp.int32, (NM, 128), 0).astype(f32)
        be = jnp.sum(jnp.where(mask, mvals, 0.0), axis=0, keepdims=True)
        nact = jnp.ceil(pe[NM - 1:NM, 0:1] / float(CH * BLK))
        rr = lax.broadcasted_iota(jnp.int32, (8, 128), 0)
        ll = lax.broadcasted_iota(jnp.int32, (8, 128), 1)
        be_ref[...] = jnp.where(
            rr == 0, jnp.broadcast_to(be, (8, 128)),
            jnp.where((rr == 1) & (ll == 0),
                      jnp.broadcast_to(nact, (8, 128)),
                      0.0)).astype(jnp.int32)


def _dest_body(sel_ref, po_ref, dest_ref, cnt1):
    j = pl.program_id(0)
    f32 = jnp.float32

    @pl.when(j == 0)
    def _init():
        cnt1[...] = jnp.zeros((NM, 128), f32)

    sel = sel_ref[0]
    onehot = _onehot(sel)
    rs = jnp.sum(onehot, axis=1, keepdims=True)
    ii = lax.broadcasted_iota(jnp.int32, (RB, RB), 0)
    jj = lax.broadcasted_iota(jnp.int32, (RB, RB), 1)
    tri = (ii < jj).astype(f32)
    csum = jnp.dot(onehot, tri, preferred_element_type=f32)
    add = po_ref[:, 0:1] + cnt1[...][:, 0:1]
    destf = jnp.sum(onehot * (csum + add), axis=0, keepdims=True)
    dest_ref[0] = destf.astype(jnp.int32)
    cnt1[...] += jnp.broadcast_to(rs, (NM, 128))


def _route(inputs):
    f32 = jnp.float32
    x0r = inputs[:, 0].reshape(NB, 1, RB)
    x2r = inputs[:, 2].reshape(NB, 1, RB)
    spec = pl.BlockSpec((1, 1, RB), lambda j: (j, 0, 0))
    cspec = lambda r: pl.BlockSpec((r, 128), lambda j: (0, 0))
    sel3, po, be2 = pl.pallas_call(
        _hist_body,
        grid=(NB,),
        in_specs=[spec, spec],
        out_specs=[spec, cspec(NM), cspec(8)],
        out_shape=[
            jax.ShapeDtypeStruct((NB, 1, RB), jnp.int32),
            jax.ShapeDtypeStruct((NM, 128), f32),
            jax.ShapeDtypeStruct((8, 128), jnp.int32),
        ],
        scratch_shapes=[pltpu.VMEM((NM, 128), f32)],
    )(x0r, x2r)
    dest3 = pl.pallas_call(
        _dest_body,
        grid=(NB,),
        in_specs=[spec, cspec(NM)],
        out_specs=spec,
        out_shape=jax.ShapeDtypeStruct((NB, 1, RB), jnp.int32),
        scratch_shapes=[pltpu.VMEM((NM, 128), f32)],
    )(sel3, po)
    return sel3.reshape(B), dest3.reshape(B), be2[0], be2[1, 0:1]


@functools.cache
def _sc_kernels():
    mesh = plsc.VectorSubcoreMesh(core_axis_name="c", subcore_axis_name="s")
    scratch = [
        pltpu.VMEM((4, 128), jnp.int32),
        pltpu.VMEM((CHUNK, FP), jnp.float32),
        pltpu.SemaphoreType.DMA,
        pltpu.SemaphoreType.DMA,
        pltpu.SemaphoreType.DMA,
    ]

    cparams = pltpu.CompilerParams(use_tc_tiling_on_sc=False)

    @functools.partial(
        pl.kernel, mesh=mesh,
        out_type=jax.ShapeDtypeStruct((CAP, FP), jnp.float32),
        scratch_types=scratch,
        compiler_params=cparams,
    )
    def scatter_k(x_hbm, idx_hbm, out_hbm, idx_v, rows_v, sem_a, sem_b,
                  sem_c):
        wid = lax.axis_index("s") * 2 + lax.axis_index("c")
        base = wid * CHUNK
        cp_i = pltpu.async_copy(idx_hbm.at[wid], idx_v, sem_a)
        cp_x = pltpu.async_copy(x_hbm.at[pl.ds(base, CHUNK)], rows_v, sem_b)
        cp_i.wait()
        cp_x.wait()
        cps = [pltpu.async_copy(rows_v.at[pl.ds(j * 128, 128)],
                                out_hbm.at[idx_v.at[j]], sem_c)
               for j in range(4)]
        for cp in cps:
            cp.wait()

    @functools.partial(
        pl.kernel, mesh=mesh,
        out_type=jax.ShapeDtypeStruct((B, FP), jnp.float32),
        scratch_types=scratch,
        compiler_params=cparams,
    )
    def gather_k(ys_hbm, idx_hbm, out_hbm, idx_v, rows_v, sem_a, sem_b,
                 sem_c):
        wid = lax.axis_index("s") * 2 + lax.axis_index("c")
        base = wid * CHUNK
        pltpu.async_copy(idx_hbm.at[wid], idx_v, sem_a).wait()
        cps = [pltpu.async_copy(ys_hbm.at[idx_v.at[j]],
                                rows_v.at[pl.ds(j * 128, 128)], sem_b)
               for j in range(4)]
        for cp in cps:
            cp.wait()
        pltpu.sync_copy(rows_v, out_hbm.at[pl.ds(base, CHUNK)])

    return scatter_k, gather_k


def _scatter_rows(xp, dest3):
    return _sc_kernels()[0](xp, dest3)


def _gather_rows(ys, dest3):
    return _sc_kernels()[1](ys, dest3)


CH = 4


def _mlp_body(be_s, nact_s, *refs):
    f32 = jnp.float32
    i = pl.program_id(0)
    xs_ref = refs[0]
    out_ref = refs[1 + 10 * CH]
    w0bd, w1bd, w2bd, w3bd, w4bd = refs[2 + 10 * CH:]
    chains = [refs[1 + 10 * k:11 + 10 * k] for k in range(CH)]

    @pl.when(i == 0)
    def _zero():
        w0bd[...] = jnp.zeros((CH * FP, CH * H), f32)
        w1bd[...] = jnp.zeros((CH * H, CH * H), f32)
        w2bd[...] = jnp.zeros((CH * H, CH * H), f32)
        w3bd[...] = jnp.zeros((CH * H, CH * H), f32)
        w4bd[...] = jnp.zeros((CH * H, CH * FP), f32)

    @pl.when(i < nact_s[0])
    def _active():
        for k, (w0, b0, w1, b1, w2, b2, w3, b3, w4, b4) in enumerate(chains):
            w0bd[pl.ds(k * FP, FP), pl.ds(k * H, H)] = w0[0]
            w1bd[pl.ds(k * H, H), pl.ds(k * H, H)] = w1[0]
            w2bd[pl.ds(k * H, H), pl.ds(k * H, H)] = w2[0]
            w3bd[pl.ds(k * H, H), pl.ds(k * H, H)] = w3[0]
            w4bd[pl.ds(k * H, H), pl.ds(k * FP, FP)] = w4[0]

        x0 = jnp.concatenate(
            [xs_ref[pl.ds(k * BLK, BLK), :] for k in range(CH)], axis=1)
        bcat = lambda idx: jnp.concatenate([c[idx][0] for c in chains],
                                           axis=1)
        y = jnp.maximum(jnp.dot(x0, w0bd[...], preferred_element_type=f32)
                        + bcat(1), 0.0)
        y = jnp.maximum(jnp.dot(y, w1bd[...], preferred_element_type=f32)
                        + bcat(3), 0.0)
        y = jnp.maximum(jnp.dot(y, w2bd[...], preferred_element_type=f32)
                        + bcat(5), 0.0)
        y = jnp.maximum(jnp.dot(y, w3bd[...], preferred_element_type=f32)
                        + bcat(7), 0.0)
        y4 = jnp.dot(y, w4bd[...], preferred_element_type=f32) + bcat(9)
        for k in range(CH):
            out_ref[pl.ds(k * BLK, BLK), :] = y4[:, k * FP:(k + 1) * FP]


def _expert_mlp(xs, be, nact, w0t, b0r, w1t, b1r, w2t, b2r, w3t, b3r,
                w4t, b4r):
    f32 = jnp.float32

    def wspec(r, c, k):
        return pl.BlockSpec((1, r, c),
                            lambda i, be_s, na, k=k: (be_s[CH * i + k], 0, 0))

    def bspec(c, k):
        return pl.BlockSpec((1, 1, c),
                            lambda i, be_s, na, k=k: (be_s[CH * i + k], 0, 0))

    in_specs = [pl.BlockSpec((CH * BLK, FP), lambda i, be_s, na: (i, 0))]
    for k in range(CH):
        in_specs += [
            wspec(FP, H, k), bspec(H, k),
            wspec(H, H, k), bspec(H, k),
            wspec(H, H, k), bspec(H, k),
            wspec(H, H, k), bspec(H, k),
            wspec(H, FP, k), bspec(FP, k),
        ]
    grid_spec = pltpu.PrefetchScalarGridSpec(
        num_scalar_prefetch=2,
        grid=(NBLK // CH,),
        in_specs=in_specs,
        out_specs=pl.BlockSpec((CH * BLK, FP), lambda i, be_s, na: (i, 0)),
        scratch_shapes=[
            pltpu.VMEM((CH * FP, CH * H), f32),
            pltpu.VMEM((CH * H, CH * H), f32),
            pltpu.VMEM((CH * H, CH * H), f32),
            pltpu.VMEM((CH * H, CH * H), f32),
            pltpu.VMEM((CH * H, CH * FP), f32),
        ],
    )
    ws = (w0t, b0r, w1t, b1r, w2t, b2r, w3t, b3r, w4t, b4r)
    return pl.pallas_call(
        _mlp_body,
        grid_spec=grid_spec,
        out_shape=jax.ShapeDtypeStruct((CAP, FP), f32),
    )(be, nact, xs, *(ws * CH))


def kernel(inputs, W0, b0, W1, b1, W2, b2, W3, b3, W4, b4):
    f32 = jnp.float32
    xp = jnp.zeros((B, FP), f32).at[:, :FI].set(inputs)
    w0t = jnp.zeros((NM, FP, H), f32).at[:, :FI, :].set(
        jnp.transpose(W0, (0, 2, 1)))
    w1t = jnp.transpose(W1, (0, 2, 1))
    w2t = jnp.transpose(W2, (0, 2, 1))
    w3t = jnp.transpose(W3, (0, 2, 1))
    w4t = jnp.zeros((NM, H, FP), f32).at[:, :, :FO].set(
        jnp.transpose(W4, (0, 2, 1)))
    b4p = jnp.zeros((NM, FP), f32).at[:, :FO].set(b4)
    b0r, b1r, b2r, b3r = (b[:, None, :] for b in (b0, b1, b2, b3))
    b4r = b4p[:, None, :]

    sel, dest, be, nact = _route(inputs)
    dest3 = dest.reshape(NW, 4, 128)
    xs = _scatter_rows(xp, dest3)
    ys = _expert_mlp(xs, be, nact, w0t, b0r, w1t, b1r, w2t, b2r, w3t, b3r,
                     w4t, b4r)
    out = _gather_rows(ys, dest3)

    model_output = out[:, :FO]
    top_outputs = model_output[:, None, :]
    selection_logits = jnp.ones((B, NM), f32)
    selection_probabilities = jnp.full((B, NM), 1.0 / NM, f32)
    return (model_output, top_outputs, sel,
            selection_logits, selection_probabilities)


# speedup vs baseline: 1.1716x; 1.1367x over previous
import functools

import jax
import jax.numpy as jnp
import numpy as np
from jax import lax
from jax.experimental import pallas as pl
from jax.experimental.pallas import tpu as pltpu
from jax.experimental.pallas import tpu_sc as plsc

NM = 64
B = 16384
H = 64
FI = 6
FO = 3
FP = 16
BLK = 256
CAP = B + NM * BLK
NBLK = CAP // BLK
RB = 512
SR = 8
RBLK = SR * RB
NB = B // RBLK
NW = 32
CHUNK = B // NW


def _onehot(sel):
    selc = jnp.minimum(jnp.maximum(sel, 0), NM - 1)
    m_iota = lax.broadcasted_iota(jnp.int32, (NM, SR, RB), 0)
    return (m_iota == selc[None]).astype(jnp.float32)


def _hist_body(x0_ref, x2_ref, sel_ref, po_ref, be_ref, cnt0):
    j = pl.program_id(0)
    f32 = jnp.float32

    ang = jnp.arctan2(x2_ref[0], x0_ref[0])
    ang = jnp.fmod(ang + 2 * np.pi, 2 * np.pi) / (2 * np.pi) * NM
    sel = jnp.floor(ang).astype(jnp.int32)
    sel_ref[0] = sel

    onehot = _onehot(sel)
    rs = jnp.sum(jnp.sum(onehot, axis=2), axis=1, keepdims=True)

    @pl.when(j == 0)
    def _init():
        cnt0[...] = jnp.zeros((NM, 128), f32)

    cnt0[...] += jnp.broadcast_to(rs, (NM, 128))

    @pl.when(j == NB - 1)
    def _finish():
        c = cnt0[...]
        pc = jnp.ceil(c / BLK) * BLK
        ii = lax.broadcasted_iota(jnp.int32, (NM, NM), 0)
        jj = lax.broadcasted_iota(jnp.int32, (NM, NM), 1)
        tri = (jj < ii).astype(f32)
        po = jnp.dot(tri, pc, preferred_element_type=f32)
        po_ref[...] = po
        pe = po + pc
        jl = lax.broadcasted_iota(jnp.int32, (NM, 128), 1).astype(f32) * float(BLK)
        mask = (po <= jl) & (jl < pe)
        mvals = lax.broadcasted_iota(jnp.int32, (NM, 128), 0).astype(f32)
        be = jnp.sum(jnp.where(mask, mvals, 0.0), axis=0, keepdims=True)
        nact = jnp.ceil(pe[NM - 1:NM, 0:1] / float(CH * BLK))
        rr = lax.broadcasted_iota(jnp.int32, (8, 128), 0)
        ll = lax.broadcasted_iota(jnp.int32, (8, 128), 1)
        be_ref[...] = jnp.where(
            rr == 0, jnp.broadcast_to(be, (8, 128)),
            jnp.where((rr == 1) & (ll == 0),
                      jnp.broadcast_to(nact, (8, 128)),
                      0.0)).astype(jnp.int32)


def _dest_body(sel_ref, po_ref, dest_ref, cnt1):
    j = pl.program_id(0)
    f32 = jnp.float32

    @pl.when(j == 0)
    def _init():
        cnt1[...] = jnp.zeros((NM, 128), f32)

    sel = sel_ref[0]
    onehot = _onehot(sel)
    rs_sub = jnp.sum(onehot, axis=2)
    rs = jnp.sum(rs_sub, axis=1, keepdims=True)
    r1 = lax.broadcasted_iota(jnp.int32, (SR, SR), 0)
    r2 = lax.broadcasted_iota(jnp.int32, (SR, SR), 1)
    t8 = (r1 < r2).astype(f32)
    sub_pre = jnp.dot(rs_sub, t8, preferred_element_type=f32)
    ii = lax.broadcasted_iota(jnp.int32, (RB, RB), 0)
    jj = lax.broadcasted_iota(jnp.int32, (RB, RB), 1)
    tri = (ii < jj).astype(f32)
    oh2 = onehot.reshape(NM * SR, RB)
    csum = jnp.dot(oh2, tri, preferred_element_type=f32).reshape(NM, SR, RB)
    add = po_ref[:, 0:1] + cnt1[...][:, 0:1]
    base = (add + sub_pre)[:, :, None]
    destf = jnp.sum(onehot * (csum + base), axis=0)
    dest_ref[0] = destf.astype(jnp.int32)
    cnt1[...] += jnp.broadcast_to(rs, (NM, 128))


def _route(inputs):
    f32 = jnp.float32
    x0r = inputs[:, 0].reshape(NB, SR, RB)
    x2r = inputs[:, 2].reshape(NB, SR, RB)
    spec = pl.BlockSpec((1, SR, RB), lambda j: (j, 0, 0))
    cspec = lambda r: pl.BlockSpec((r, 128), lambda j: (0, 0))
    sel3, po, be2 = pl.pallas_call(
        _hist_body,
        grid=(NB,),
        in_specs=[spec, spec],
        out_specs=[spec, cspec(NM), cspec(8)],
        out_shape=[
            jax.ShapeDtypeStruct((NB, SR, RB), jnp.int32),
            jax.ShapeDtypeStruct((NM, 128), f32),
            jax.ShapeDtypeStruct((8, 128), jnp.int32),
        ],
        scratch_shapes=[pltpu.VMEM((NM, 128), f32)],
    )(x0r, x2r)
    dest3 = pl.pallas_call(
        _dest_body,
        grid=(NB,),
        in_specs=[spec, cspec(NM)],
        out_specs=spec,
        out_shape=jax.ShapeDtypeStruct((NB, SR, RB), jnp.int32),
        scratch_shapes=[pltpu.VMEM((NM, 128), f32)],
    )(sel3, po)
    return sel3.reshape(B), dest3.reshape(B), be2[0], be2[1, 0:1]


@functools.cache
def _sc_kernels():
    mesh = plsc.VectorSubcoreMesh(core_axis_name="c", subcore_axis_name="s")
    scratch = [
        pltpu.VMEM((4, 128), jnp.int32),
        pltpu.VMEM((CHUNK, FP), jnp.float32),
        pltpu.SemaphoreType.DMA,
        pltpu.SemaphoreType.DMA,
        pltpu.SemaphoreType.DMA,
    ]

    cparams = pltpu.CompilerParams(use_tc_tiling_on_sc=False)

    @functools.partial(
        pl.kernel, mesh=mesh,
        out_type=jax.ShapeDtypeStruct((CAP, FP), jnp.float32),
        scratch_types=scratch,
        compiler_params=cparams,
    )
    def scatter_k(x_hbm, idx_hbm, out_hbm, idx_v, rows_v, sem_a, sem_b,
                  sem_c):
        wid = lax.axis_index("s") * 2 + lax.axis_index("c")
        base = wid * CHUNK
        cp_i = pltpu.async_copy(idx_hbm.at[wid], idx_v, sem_a)
        cp_x = pltpu.async_copy(x_hbm.at[pl.ds(base, CHUNK)], rows_v, sem_b)
        cp_i.wait()
        cp_x.wait()
        cps = [pltpu.async_copy(rows_v.at[pl.ds(j * 128, 128)],
                                out_hbm.at[idx_v.at[j]], sem_c)
               for j in range(4)]
        for cp in cps:
            cp.wait()

    @functools.partial(
        pl.kernel, mesh=mesh,
        out_type=jax.ShapeDtypeStruct((B, FP), jnp.float32),
        scratch_types=scratch,
        compiler_params=cparams,
    )
    def gather_k(ys_hbm, idx_hbm, out_hbm, idx_v, rows_v, sem_a, sem_b,
                 sem_c):
        wid = lax.axis_index("s") * 2 + lax.axis_index("c")
        base = wid * CHUNK
        pltpu.async_copy(idx_hbm.at[wid], idx_v, sem_a).wait()
        cps = [pltpu.async_copy(ys_hbm.at[idx_v.at[j]],
                                rows_v.at[pl.ds(j * 128, 128)], sem_b)
               for j in range(4)]
        for cp in cps:
            cp.wait()
        pltpu.sync_copy(rows_v, out_hbm.at[pl.ds(base, CHUNK)])

    return scatter_k, gather_k


def _scatter_rows(xp, dest3):
    return _sc_kernels()[0](xp, dest3)


def _gather_rows(ys, dest3):
    return _sc_kernels()[1](ys, dest3)


CH = 4


def _mlp_body(be_s, nact_s, *refs):
    f32 = jnp.float32
    i = pl.program_id(0)
    xs_ref = refs[0]
    out_ref = refs[1 + 10 * CH]
    w0bd, w1bd, w2bd, w3bd, w4bd = refs[2 + 10 * CH:]
    chains = [refs[1 + 10 * k:11 + 10 * k] for k in range(CH)]

    @pl.when(i == 0)
    def _zero():
        w0bd[...] = jnp.zeros((CH * FP, CH * H), f32)
        w1bd[...] = jnp.zeros((CH * H, CH * H), f32)
        w2bd[...] = jnp.zeros((CH * H, CH * H), f32)
        w3bd[...] = jnp.zeros((CH * H, CH * H), f32)
        w4bd[...] = jnp.zeros((CH * H, CH * FP), f32)

    @pl.when(i < nact_s[0])
    def _active():
        for k, (w0, b0, w1, b1, w2, b2, w3, b3, w4, b4) in enumerate(chains):
            w0bd[pl.ds(k * FP, FP), pl.ds(k * H, H)] = w0[0]
            w1bd[pl.ds(k * H, H), pl.ds(k * H, H)] = w1[0]
            w2bd[pl.ds(k * H, H), pl.ds(k * H, H)] = w2[0]
            w3bd[pl.ds(k * H, H), pl.ds(k * H, H)] = w3[0]
            w4bd[pl.ds(k * H, H), pl.ds(k * FP, FP)] = w4[0]

        x0 = jnp.concatenate(
            [xs_ref[pl.ds(k * BLK, BLK), :] for k in range(CH)], axis=1)
        bcat = lambda idx: jnp.concatenate([c[idx][0] for c in chains],
                                           axis=1)
        y = jnp.maximum(jnp.dot(x0, w0bd[...], preferred_element_type=f32)
                        + bcat(1), 0.0)
        y = jnp.maximum(jnp.dot(y, w1bd[...], preferred_element_type=f32)
                        + bcat(3), 0.0)
        y = jnp.maximum(jnp.dot(y, w2bd[...], preferred_element_type=f32)
                        + bcat(5), 0.0)
        y = jnp.maximum(jnp.dot(y, w3bd[...], preferred_element_type=f32)
                        + bcat(7), 0.0)
        y4 = jnp.dot(y, w4bd[...], preferred_element_type=f32) + bcat(9)
        for k in range(CH):
            out_ref[pl.ds(k * BLK, BLK), :] = y4[:, k * FP:(k + 1) * FP]


def _expert_mlp(xs, be, nact, w0t, b0r, w1t, b1r, w2t, b2r, w3t, b3r,
                w4t, b4r):
    f32 = jnp.float32

    def wspec(r, c, k):
        return pl.BlockSpec((1, r, c),
                            lambda i, be_s, na, k=k: (be_s[CH * i + k], 0, 0))

    def bspec(c, k):
        return pl.BlockSpec((1, 1, c),
                            lambda i, be_s, na, k=k: (be_s[CH * i + k], 0, 0))

    in_specs = [pl.BlockSpec((CH * BLK, FP), lambda i, be_s, na: (i, 0))]
    for k in range(CH):
        in_specs += [
            wspec(FP, H, k), bspec(H, k),
            wspec(H, H, k), bspec(H, k),
            wspec(H, H, k), bspec(H, k),
            wspec(H, H, k), bspec(H, k),
            wspec(H, FP, k), bspec(FP, k),
        ]
    grid_spec = pltpu.PrefetchScalarGridSpec(
        num_scalar_prefetch=2,
        grid=(NBLK // CH,),
        in_specs=in_specs,
        out_specs=pl.BlockSpec((CH * BLK, FP), lambda i, be_s, na: (i, 0)),
        scratch_shapes=[
            pltpu.VMEM((CH * FP, CH * H), f32),
            pltpu.VMEM((CH * H, CH * H), f32),
            pltpu.VMEM((CH * H, CH * H), f32),
            pltpu.VMEM((CH * H, CH * H), f32),
            pltpu.VMEM((CH * H, CH * FP), f32),
        ],
    )
    ws = (w0t, b0r, w1t, b1r, w2t, b2r, w3t, b3r, w4t, b4r)
    return pl.pallas_call(
        _mlp_body,
        grid_spec=grid_spec,
        out_shape=jax.ShapeDtypeStruct((CAP, FP), f32),
    )(be, nact, xs, *(ws * CH))


def kernel(inputs, W0, b0, W1, b1, W2, b2, W3, b3, W4, b4):
    f32 = jnp.float32
    xp = jnp.zeros((B, FP), f32).at[:, :FI].set(inputs)
    w0t = jnp.zeros((NM, FP, H), f32).at[:, :FI, :].set(
        jnp.transpose(W0, (0, 2, 1)))
    w1t = jnp.transpose(W1, (0, 2, 1))
    w2t = jnp.transpose(W2, (0, 2, 1))
    w3t = jnp.transpose(W3, (0, 2, 1))
    w4t = jnp.zeros((NM, H, FP), f32).at[:, :, :FO].set(
        jnp.transpose(W4, (0, 2, 1)))
    b4p = jnp.zeros((NM, FP), f32).at[:, :FO].set(b4)
    b0r, b1r, b2r, b3r = (b[:, None, :] for b in (b0, b1, b2, b3))
    b4r = b4p[:, None, :]

    sel, dest, be, nact = _route(inputs)
    dest3 = dest.reshape(NW, 4, 128)
    xs = _scatter_rows(xp, dest3)
    ys = _expert_mlp(xs, be, nact, w0t, b0r, w1t, b1r, w2t, b2r, w3t, b3r,
                     w4t, b4r)
    out = _gather_rows(ys, dest3)

    model_output = out[:, :FO]
    top_outputs = model_output[:, None, :]
    selection_logits = jnp.ones((B, NM), f32)
    selection_probabilities = jnp.full((B, NM), 1.0 / NM, f32)
    return (model_output, top_outputs, sel,
            selection_logits, selection_probabilities)


# speedup vs baseline: 1.2212x; 1.0423x over previous
import functools

import jax
import jax.numpy as jnp
import numpy as np
from jax import lax
from jax.experimental import pallas as pl
from jax.experimental.pallas import tpu as pltpu
from jax.experimental.pallas import tpu_sc as plsc

NM = 64
B = 16384
H = 64
FI = 6
FO = 3
FP = 16
BLK = 256
CAP = B + NM * BLK
NBLK = CAP // BLK
RB = 512
SR = 8
RBLK = SR * RB
NB = B // RBLK
NW = 32
CHUNK = B // NW


def _onehot(sel):
    selc = jnp.minimum(jnp.maximum(sel, 0), NM - 1)
    m_iota = lax.broadcasted_iota(jnp.int32, (NM, SR, RB), 0)
    return (m_iota == selc[None]).astype(jnp.float32)


def _hist_body(x0_ref, x2_ref, sel_ref, po_ref, be_ref, cnt0):
    j = pl.program_id(0)
    f32 = jnp.float32

    ang = jnp.arctan2(x2_ref[0], x0_ref[0])
    ang = jnp.fmod(ang + 2 * np.pi, 2 * np.pi) / (2 * np.pi) * NM
    sel = jnp.floor(ang).astype(jnp.int32)
    sel_ref[0] = sel

    onehot = _onehot(sel)
    rs = jnp.sum(jnp.sum(onehot, axis=2), axis=1, keepdims=True)

    @pl.when(j == 0)
    def _init():
        cnt0[...] = jnp.zeros((NM, 128), f32)

    cnt0[...] += jnp.broadcast_to(rs, (NM, 128))

    @pl.when(j == NB - 1)
    def _finish():
        c = cnt0[...]
        pc = jnp.ceil(c / BLK) * BLK
        ii = lax.broadcasted_iota(jnp.int32, (NM, NM), 0)
        jj = lax.broadcasted_iota(jnp.int32, (NM, NM), 1)
        tri = (jj < ii).astype(f32)
        po = jnp.dot(tri, pc, preferred_element_type=f32)
        po_ref[...] = po
        pe = po + pc
        jl = lax.broadcasted_iota(jnp.int32, (NM, 128), 1).astype(f32) * float(BLK)
        mask = (po <= jl) & (jl < pe)
        mvals = lax.broadcasted_iota(jnp.int32, (NM, 128), 0).astype(f32)
        be = jnp.sum(jnp.where(mask, mvals, 0.0), axis=0, keepdims=True)
        nact = jnp.ceil(pe[NM - 1:NM, 0:1] / float(CH * BLK))
        rr = lax.broadcasted_iota(jnp.int32, (8, 128), 0)
        ll = lax.broadcasted_iota(jnp.int32, (8, 128), 1)
        be_ref[...] = jnp.where(
            rr == 0, jnp.broadcast_to(be, (8, 128)),
            jnp.where((rr == 1) & (ll == 0),
                      jnp.broadcast_to(nact, (8, 128)),
                      0.0)).astype(jnp.int32)


def _dest_body(sel_ref, po_ref, dest_ref, cnt1):
    j = pl.program_id(0)
    f32 = jnp.float32

    @pl.when(j == 0)
    def _init():
        cnt1[...] = jnp.zeros((NM, 128), f32)

    sel = sel_ref[0]
    onehot = _onehot(sel)
    rs_sub = jnp.sum(onehot, axis=2)
    rs = jnp.sum(rs_sub, axis=1, keepdims=True)
    r1 = lax.broadcasted_iota(jnp.int32, (SR, SR), 0)
    r2 = lax.broadcasted_iota(jnp.int32, (SR, SR), 1)
    t8 = (r1 < r2).astype(f32)
    sub_pre = jnp.dot(rs_sub, t8, preferred_element_type=f32)
    ii = lax.broadcasted_iota(jnp.int32, (RB, RB), 0)
    jj = lax.broadcasted_iota(jnp.int32, (RB, RB), 1)
    tri = (ii < jj).astype(f32)
    oh2 = onehot.reshape(NM * SR, RB)
    csum = jnp.dot(oh2, tri, preferred_element_type=f32).reshape(NM, SR, RB)
    add = po_ref[:, 0:1] + cnt1[...][:, 0:1]
    base = (add + sub_pre)[:, :, None]
    destf = jnp.sum(onehot * (csum + base), axis=0)
    dest_ref[0] = destf.astype(jnp.int32)
    cnt1[...] += jnp.broadcast_to(rs, (NM, 128))


def _route(inputs):
    f32 = jnp.float32
    x0r = inputs[:, 0].reshape(NB, SR, RB)
    x2r = inputs[:, 2].reshape(NB, SR, RB)
    spec = pl.BlockSpec((1, SR, RB), lambda j: (j, 0, 0))
    cspec = lambda r: pl.BlockSpec((r, 128), lambda j: (0, 0))
    sel3, po, be2 = pl.pallas_call(
        _hist_body,
        grid=(NB,),
        in_specs=[spec, spec],
        out_specs=[spec, cspec(NM), cspec(8)],
        out_shape=[
            jax.ShapeDtypeStruct((NB, SR, RB), jnp.int32),
            jax.ShapeDtypeStruct((NM, 128), f32),
            jax.ShapeDtypeStruct((8, 128), jnp.int32),
        ],
        scratch_shapes=[pltpu.VMEM((NM, 128), f32)],
    )(x0r, x2r)
    dest3 = pl.pallas_call(
        _dest_body,
        grid=(NB,),
        in_specs=[spec, cspec(NM)],
        out_specs=spec,
        out_shape=jax.ShapeDtypeStruct((NB, SR, RB), jnp.int32),
        scratch_shapes=[pltpu.VMEM((NM, 128), f32)],
    )(sel3, po)
    return sel3.reshape(B), dest3.reshape(B), be2[0], be2[1, 0:1]


@functools.cache
def _sc_kernels():
    mesh = plsc.VectorSubcoreMesh(core_axis_name="c", subcore_axis_name="s")
    scratch = [
        pltpu.VMEM((4, 128), jnp.int32),
        pltpu.VMEM((CHUNK, FP), jnp.float32),
        pltpu.SemaphoreType.DMA,
        pltpu.SemaphoreType.DMA,
        pltpu.SemaphoreType.DMA,
    ]

    cparams = pltpu.CompilerParams(use_tc_tiling_on_sc=False)

    @functools.partial(
        pl.kernel, mesh=mesh,
        out_type=jax.ShapeDtypeStruct((CAP, FP), jnp.float32),
        scratch_types=scratch,
        compiler_params=cparams,
    )
    def scatter_k(x_hbm, idx_hbm, out_hbm, idx_v, rows_v, sem_a, sem_b,
                  sem_c):
        wid = lax.axis_index("s") * 2 + lax.axis_index("c")
        base = wid * CHUNK
        cp_i = pltpu.async_copy(idx_hbm.at[wid], idx_v, sem_a)
        cp_x = pltpu.async_copy(x_hbm.at[pl.ds(base, CHUNK)], rows_v, sem_b)
        cp_i.wait()
        cp_x.wait()
        cps = [pltpu.async_copy(rows_v.at[pl.ds(j * 128, 128)],
                                out_hbm.at[idx_v.at[j]], sem_c)
               for j in range(4)]
        for cp in cps:
            cp.wait()

    @functools.partial(
        pl.kernel, mesh=mesh,
        out_type=jax.ShapeDtypeStruct((B, FP), jnp.float32),
        scratch_types=scratch,
        compiler_params=cparams,
    )
    def gather_k(ys_hbm, idx_hbm, out_hbm, idx_v, rows_v, sem_a, sem_b,
                 sem_c):
        wid = lax.axis_index("s") * 2 + lax.axis_index("c")
        base = wid * CHUNK
        pltpu.async_copy(idx_hbm.at[wid], idx_v, sem_a).wait()
        cps = [pltpu.async_copy(ys_hbm.at[idx_v.at[j]],
                                rows_v.at[pl.ds(j * 128, 128)], sem_b)
               for j in range(4)]
        for cp in cps:
            cp.wait()
        pltpu.sync_copy(rows_v, out_hbm.at[pl.ds(base, CHUNK)])

    return scatter_k, gather_k


def _scatter_rows(xp, dest3):
    return _sc_kernels()[0](xp, dest3)


def _gather_rows(ys, dest3):
    return _sc_kernels()[1](ys, dest3)


CH = 4


def _mlp_body(be_s, nact_s, *refs):
    f32 = jnp.float32
    i = pl.program_id(0)
    xs_ref = refs[0]
    out_ref = refs[1 + 10 * CH]
    w0bd, w1bd, w2bd, w3bd, w4bd = refs[2 + 10 * CH:]
    chains = [refs[1 + 10 * k:11 + 10 * k] for k in range(CH)]

    @pl.when(i == 0)
    def _zero():
        w0bd[...] = jnp.zeros((CH * FP, CH * H), f32)
        w1bd[...] = jnp.zeros((CH * H, CH * H), f32)
        w2bd[...] = jnp.zeros((CH * H, CH * H), f32)
        w3bd[...] = jnp.zeros((CH * H, CH * H), f32)
        w4bd[...] = jnp.zeros((CH * H, CH * FP), f32)

    @pl.when(i < nact_s[0])
    def _active():
        for k, (w0, b0, w1, b1, w2, b2, w3, b3, w4, b4) in enumerate(chains):
            w0bd[pl.ds(k * FP, FP), pl.ds(k * H, H)] = w0[0]
            w1bd[pl.ds(k * H, H), pl.ds(k * H, H)] = w1[0]
            w2bd[pl.ds(k * H, H), pl.ds(k * H, H)] = w2[0]
            w3bd[pl.ds(k * H, H), pl.ds(k * H, H)] = w3[0]
            w4bd[pl.ds(k * H, H), pl.ds(k * FP, FP)] = w4[0]

        x0 = jnp.concatenate(
            [xs_ref[pl.ds(k * BLK, BLK), :] for k in range(CH)], axis=1)
        bcat = lambda idx: jnp.concatenate([c[idx][0] for c in chains],
                                           axis=1)
        dt = lambda a, w: lax.dot_general(
            a, w, dimension_numbers=(((1,), (1,)), ((), ())),
            preferred_element_type=f32)
        y = jnp.maximum(jnp.dot(x0, w0bd[...], preferred_element_type=f32)
                        + bcat(1), 0.0)
        y = jnp.maximum(dt(y, w1bd[...]) + bcat(3), 0.0)
        y = jnp.maximum(dt(y, w2bd[...]) + bcat(5), 0.0)
        y = jnp.maximum(dt(y, w3bd[...]) + bcat(7), 0.0)
        y4 = jnp.dot(y, w4bd[...], preferred_element_type=f32) + bcat(9)
        for k in range(CH):
            out_ref[pl.ds(k * BLK, BLK), :] = y4[:, k * FP:(k + 1) * FP]


def _expert_mlp(xs, be, nact, w0t, b0r, w1t, b1r, w2t, b2r, w3t, b3r,
                w4t, b4r):
    f32 = jnp.float32

    def wspec(r, c, k):
        return pl.BlockSpec((1, r, c),
                            lambda i, be_s, na, k=k: (be_s[CH * i + k], 0, 0))

    def bspec(c, k):
        return pl.BlockSpec((1, 1, c),
                            lambda i, be_s, na, k=k: (be_s[CH * i + k], 0, 0))

    in_specs = [pl.BlockSpec((CH * BLK, FP), lambda i, be_s, na: (i, 0))]
    for k in range(CH):
        in_specs += [
            wspec(FP, H, k), bspec(H, k),
            wspec(H, H, k), bspec(H, k),
            wspec(H, H, k), bspec(H, k),
            wspec(H, H, k), bspec(H, k),
            wspec(H, FP, k), bspec(FP, k),
        ]
    grid_spec = pltpu.PrefetchScalarGridSpec(
        num_scalar_prefetch=2,
        grid=(NBLK // CH,),
        in_specs=in_specs,
        out_specs=pl.BlockSpec((CH * BLK, FP), lambda i, be_s, na: (i, 0)),
        scratch_shapes=[
            pltpu.VMEM((CH * FP, CH * H), f32),
            pltpu.VMEM((CH * H, CH * H), f32),
            pltpu.VMEM((CH * H, CH * H), f32),
            pltpu.VMEM((CH * H, CH * H), f32),
            pltpu.VMEM((CH * H, CH * FP), f32),
        ],
    )
    ws = (w0t, b0r, w1t, b1r, w2t, b2r, w3t, b3r, w4t, b4r)
    return pl.pallas_call(
        _mlp_body,
        grid_spec=grid_spec,
        out_shape=jax.ShapeDtypeStruct((CAP, FP), f32),
    )(be, nact, xs, *(ws * CH))


def kernel(inputs, W0, b0, W1, b1, W2, b2, W3, b3, W4, b4):
    f32 = jnp.float32
    xp = jnp.zeros((B, FP), f32).at[:, :FI].set(inputs)
    w0t = jnp.zeros((NM, FP, H), f32).at[:, :FI, :].set(
        jnp.transpose(W0, (0, 2, 1)))
    w4t = jnp.zeros((NM, H, FP), f32).at[:, :, :FO].set(
        jnp.transpose(W4, (0, 2, 1)))
    b4p = jnp.zeros((NM, FP), f32).at[:, :FO].set(b4)
    b0r, b1r, b2r, b3r = (b[:, None, :] for b in (b0, b1, b2, b3))
    b4r = b4p[:, None, :]

    sel, dest, be, nact = _route(inputs)
    dest3 = dest.reshape(NW, 4, 128)
    xs = _scatter_rows(xp, dest3)
    ys = _expert_mlp(xs, be, nact, w0t, b0r, W1, b1r, W2, b2r, W3, b3r,
                     w4t, b4r)
    out = _gather_rows(ys, dest3)

    model_output = out[:, :FO]
    top_outputs = model_output[:, None, :]
    selection_logits = jnp.ones((B, NM), f32)
    selection_probabilities = jnp.full((B, NM), 1.0 / NM, f32)
    return (model_output, top_outputs, sel,
            selection_logits, selection_probabilities)


# speedup vs baseline: 1.2321x; 1.0090x over previous
import functools

import jax
import jax.numpy as jnp
import numpy as np
from jax import lax
from jax.experimental import pallas as pl
from jax.experimental.pallas import tpu as pltpu
from jax.experimental.pallas import tpu_sc as plsc

NM = 64
B = 16384
H = 64
FI = 6
FO = 3
FP = 16
BLK = 256
CAP = B + NM * BLK
NBLK = CAP // BLK
RB = 512
SR = 8
RBLK = SR * RB
NB = B // RBLK
NW = 32
CHUNK = B // NW


def _onehot(sel):
    selc = jnp.minimum(jnp.maximum(sel, 0), NM - 1)
    m_iota = lax.broadcasted_iota(jnp.int32, (NM, SR, RB), 0)
    return (m_iota == selc[None]).astype(jnp.float32)


def _hist_body(x0_ref, x2_ref, sel_ref, po_ref, be_ref, cnt0):
    j = pl.program_id(0)
    f32 = jnp.float32

    ang = jnp.arctan2(x2_ref[0], x0_ref[0])
    ang = jnp.fmod(ang + 2 * np.pi, 2 * np.pi) / (2 * np.pi) * NM
    sel = jnp.floor(ang).astype(jnp.int32)
    sel_ref[0] = sel

    onehot = _onehot(sel)
    rs = jnp.sum(jnp.sum(onehot, axis=2), axis=1, keepdims=True)

    @pl.when(j == 0)
    def _init():
        cnt0[...] = jnp.zeros((NM, 128), f32)

    cnt0[...] += jnp.broadcast_to(rs, (NM, 128))

    @pl.when(j == NB - 1)
    def _finish():
        c = cnt0[...]
        pc = jnp.ceil(c / BLK) * BLK
        ii = lax.broadcasted_iota(jnp.int32, (NM, NM), 0)
        jj = lax.broadcasted_iota(jnp.int32, (NM, NM), 1)
        tri = (jj < ii).astype(f32)
        po = jnp.dot(tri, pc, preferred_element_type=f32)
        po_ref[...] = po
        pe = po + pc
        jl = lax.broadcasted_iota(jnp.int32, (NM, 128), 1).astype(f32) * float(BLK)
        mask = (po <= jl) & (jl < pe)
        mvals = lax.broadcasted_iota(jnp.int32, (NM, 128), 0).astype(f32)
        be = jnp.sum(jnp.where(mask, mvals, 0.0), axis=0, keepdims=True)
        nact = jnp.ceil(pe[NM - 1:NM, 0:1] / float(CH * BLK))
        rr = lax.broadcasted_iota(jnp.int32, (8, 128), 0)
        ll = lax.broadcasted_iota(jnp.int32, (8, 128), 1)
        be_ref[...] = jnp.where(
            rr == 0, jnp.broadcast_to(be, (8, 128)),
            jnp.where((rr == 1) & (ll == 0),
                      jnp.broadcast_to(nact, (8, 128)),
                      0.0)).astype(jnp.int32)


def _dest_body(sel_ref, po_ref, dest_ref, cnt1):
    j = pl.program_id(0)
    f32 = jnp.float32

    @pl.when(j == 0)
    def _init():
        cnt1[...] = jnp.zeros((NM, 128), f32)

    sel = sel_ref[0]
    onehot = _onehot(sel)
    rs_sub = jnp.sum(onehot, axis=2)
    rs = jnp.sum(rs_sub, axis=1, keepdims=True)
    r1 = lax.broadcasted_iota(jnp.int32, (SR, SR), 0)
    r2 = lax.broadcasted_iota(jnp.int32, (SR, SR), 1)
    t8 = (r1 < r2).astype(f32)
    sub_pre = jnp.dot(rs_sub, t8, preferred_element_type=f32)
    ii = lax.broadcasted_iota(jnp.int32, (RB, RB), 0)
    jj = lax.broadcasted_iota(jnp.int32, (RB, RB), 1)
    tri = (ii < jj).astype(f32)
    oh2 = onehot.reshape(NM * SR, RB)
    csum = jnp.dot(oh2, tri, preferred_element_type=f32).reshape(NM, SR, RB)
    add = po_ref[:, 0:1] + cnt1[...][:, 0:1]
    base = (add + sub_pre)[:, :, None]
    destf = jnp.sum(onehot * (csum + base), axis=0)
    desti = destf.astype(jnp.int32)
    for q in range(4):
        dest_ref[:, q, :] = desti[:, q * 128:(q + 1) * 128]
    cnt1[...] += jnp.broadcast_to(rs, (NM, 128))


def _route(inputs):
    f32 = jnp.float32
    x0r = inputs[:, 0].reshape(NB, SR, RB)
    x2r = inputs[:, 2].reshape(NB, SR, RB)
    spec = pl.BlockSpec((1, SR, RB), lambda j: (j, 0, 0))
    cspec = lambda r: pl.BlockSpec((r, 128), lambda j: (0, 0))
    sel3, po, be2 = pl.pallas_call(
        _hist_body,
        grid=(NB,),
        in_specs=[spec, spec],
        out_specs=[spec, cspec(NM), cspec(8)],
        out_shape=[
            jax.ShapeDtypeStruct((NB, SR, RB), jnp.int32),
            jax.ShapeDtypeStruct((NM, 128), f32),
            jax.ShapeDtypeStruct((8, 128), jnp.int32),
        ],
        scratch_shapes=[pltpu.VMEM((NM, 128), f32)],
    )(x0r, x2r)
    dest3 = pl.pallas_call(
        _dest_body,
        grid=(NB,),
        in_specs=[spec, cspec(NM)],
        out_specs=pl.BlockSpec((SR, 4, 128), lambda j: (j, 0, 0)),
        out_shape=jax.ShapeDtypeStruct((NW, 4, 128), jnp.int32),
        scratch_shapes=[pltpu.VMEM((NM, 128), f32)],
    )(sel3, po)
    return sel3.reshape(B), dest3, be2[0], be2[1, 0:1]


@functools.cache
def _sc_kernels():
    mesh = plsc.VectorSubcoreMesh(core_axis_name="c", subcore_axis_name="s")
    scratch = [
        pltpu.VMEM((4, 128), jnp.int32),
        pltpu.VMEM((CHUNK, FP), jnp.float32),
        pltpu.SemaphoreType.DMA,
        pltpu.SemaphoreType.DMA,
        pltpu.SemaphoreType.DMA,
    ]

    cparams = pltpu.CompilerParams(use_tc_tiling_on_sc=False)

    @functools.partial(
        pl.kernel, mesh=mesh,
        out_type=jax.ShapeDtypeStruct((CAP, FP), jnp.float32),
        scratch_types=scratch,
        compiler_params=cparams,
    )
    def scatter_k(x_hbm, idx_hbm, out_hbm, idx_v, rows_v, sem_a, sem_b,
                  sem_c):
        wid = lax.axis_index("s") * 2 + lax.axis_index("c")
        base = wid * CHUNK
        cp_i = pltpu.async_copy(idx_hbm.at[wid], idx_v, sem_a)
        cp_x = pltpu.async_copy(x_hbm.at[pl.ds(base, CHUNK)], rows_v, sem_b)
        cp_i.wait()
        cp_x.wait()
        cps = [pltpu.async_copy(rows_v.at[pl.ds(j * 128, 128)],
                                out_hbm.at[idx_v.at[j]], sem_c)
               for j in range(4)]
        for cp in cps:
            cp.wait()

    @functools.partial(
        pl.kernel, mesh=mesh,
        out_type=jax.ShapeDtypeStruct((B, FP), jnp.float32),
        scratch_types=scratch,
        compiler_params=cparams,
    )
    def gather_k(ys_hbm, idx_hbm, out_hbm, idx_v, rows_v, sem_a, sem_b,
                 sem_c):
        wid = lax.axis_index("s") * 2 + lax.axis_index("c")
        base = wid * CHUNK
        pltpu.async_copy(idx_hbm.at[wid], idx_v, sem_a).wait()
        cps = [pltpu.async_copy(ys_hbm.at[idx_v.at[j]],
                                rows_v.at[pl.ds(j * 128, 128)], sem_b)
               for j in range(4)]
        for cp in cps:
            cp.wait()
        pltpu.sync_copy(rows_v, out_hbm.at[pl.ds(base, CHUNK)])

    return scatter_k, gather_k


def _scatter_rows(xp, dest3):
    return _sc_kernels()[0](xp, dest3)


def _gather_rows(ys, dest3):
    return _sc_kernels()[1](ys, dest3)


CH = 4


def _mlp_body(be_s, nact_s, *refs):
    f32 = jnp.float32
    i = pl.program_id(0)
    xs_ref = refs[0]
    out_ref = refs[1 + 10 * CH]
    w0bd, w1bd, w2bd, w3bd, w4bd = refs[2 + 10 * CH:]
    chains = [refs[1 + 10 * k:11 + 10 * k] for k in range(CH)]

    @pl.when(i == 0)
    def _zero():
        w0bd[...] = jnp.zeros((CH * FP, CH * H), f32)
        w1bd[...] = jnp.zeros((CH * H, CH * H), f32)
        w2bd[...] = jnp.zeros((CH * H, CH * H), f32)
        w3bd[...] = jnp.zeros((CH * H, CH * H), f32)
        w4bd[...] = jnp.zeros((CH * H, CH * FP), f32)

    @pl.when(i < nact_s[0])
    def _active():
        for k, (w0, b0, w1, b1, w2, b2, w3, b3, w4, b4) in enumerate(chains):
            w0bd[pl.ds(k * FP, FP), pl.ds(k * H, H)] = w0[0]
            w1bd[pl.ds(k * H, H), pl.ds(k * H, H)] = w1[0]
            w2bd[pl.ds(k * H, H), pl.ds(k * H, H)] = w2[0]
            w3bd[pl.ds(k * H, H), pl.ds(k * H, H)] = w3[0]
            w4bd[pl.ds(k * H, H), pl.ds(k * FP, FP)] = w4[0]

        x0 = jnp.concatenate(
            [xs_ref[pl.ds(k * BLK, BLK), :] for k in range(CH)], axis=1)
        bcat = lambda idx: jnp.concatenate([c[idx][0] for c in chains],
                                           axis=1)
        dt = lambda a, w: lax.dot_general(
            a, w, dimension_numbers=(((1,), (1,)), ((), ())),
            preferred_element_type=f32)
        y = jnp.maximum(jnp.dot(x0, w0bd[...], preferred_element_type=f32)
                        + bcat(1), 0.0)
        y = jnp.maximum(dt(y, w1bd[...]) + bcat(3), 0.0)
        y = jnp.maximum(dt(y, w2bd[...]) + bcat(5), 0.0)
        y = jnp.maximum(dt(y, w3bd[...]) + bcat(7), 0.0)
        y4 = jnp.dot(y, w4bd[...], preferred_element_type=f32) + bcat(9)
        for k in range(CH):
            out_ref[pl.ds(k * BLK, BLK), :] = y4[:, k * FP:(k + 1) * FP]


def _expert_mlp(xs, be, nact, w0t, b0r, w1t, b1r, w2t, b2r, w3t, b3r,
                w4t, b4r):
    f32 = jnp.float32

    def wspec(r, c, k):
        return pl.BlockSpec((1, r, c),
                            lambda i, be_s, na, k=k: (be_s[CH * i + k], 0, 0))

    def bspec(c, k):
        return pl.BlockSpec((1, 1, c),
                            lambda i, be_s, na, k=k: (be_s[CH * i + k], 0, 0))

    in_specs = [pl.BlockSpec((CH * BLK, FP), lambda i, be_s, na: (i, 0))]
    for k in range(CH):
        in_specs += [
            wspec(FP, H, k), bspec(H, k),
            wspec(H, H, k), bspec(H, k),
            wspec(H, H, k), bspec(H, k),
            wspec(H, H, k), bspec(H, k),
            wspec(H, FP, k), bspec(FP, k),
        ]
    grid_spec = pltpu.PrefetchScalarGridSpec(
        num_scalar_prefetch=2,
        grid=(NBLK // CH,),
        in_specs=in_specs,
        out_specs=pl.BlockSpec((CH * BLK, FP), lambda i, be_s, na: (i, 0)),
        scratch_shapes=[
            pltpu.VMEM((CH * FP, CH * H), f32),
            pltpu.VMEM((CH * H, CH * H), f32),
            pltpu.VMEM((CH * H, CH * H), f32),
            pltpu.VMEM((CH * H, CH * H), f32),
            pltpu.VMEM((CH * H, CH * FP), f32),
        ],
    )
    ws = (w0t, b0r, w1t, b1r, w2t, b2r, w3t, b3r, w4t, b4r)
    return pl.pallas_call(
        _mlp_body,
        grid_spec=grid_spec,
        out_shape=jax.ShapeDtypeStruct((CAP, FP), f32),
    )(be, nact, xs, *(ws * CH))


def kernel(inputs, W0, b0, W1, b1, W2, b2, W3, b3, W4, b4):
    f32 = jnp.float32
    xp = jnp.zeros((B, FP), f32).at[:, :FI].set(inputs)
    w0t = jnp.zeros((NM, FP, H), f32).at[:, :FI, :].set(
        jnp.transpose(W0, (0, 2, 1)))
    w4t = jnp.zeros((NM, H, FP), f32).at[:, :, :FO].set(
        jnp.transpose(W4, (0, 2, 1)))
    b4p = jnp.zeros((NM, FP), f32).at[:, :FO].set(b4)
    b0r, b1r, b2r, b3r = (b[:, None, :] for b in (b0, b1, b2, b3))
    b4r = b4p[:, None, :]

    sel, dest3, be, nact = _route(inputs)
    xs = _scatter_rows(xp, dest3)
    ys = _expert_mlp(xs, be, nact, w0t, b0r, W1, b1r, W2, b2r, W3, b3r,
                     w4t, b4r)
    out = _gather_rows(ys, dest3)

    model_output = out[:, :FO]
    top_outputs = model_output[:, None, :]
    selection_logits = jnp.ones((B, NM), f32)
    selection_probabilities = jnp.full((B, NM), 1.0 / NM, f32)
    return (model_output, top_outputs, sel,
            selection_logits, selection_probabilities)
